# baseline (speedup 1.0000x reference)
import jax
import jax.numpy as jnp
from jax import lax
from jax.experimental import pallas as pl
from jax.experimental.pallas import tpu as pltpu
from jax.experimental.pallas import tpu_sc as plsc

_B = 16384
_F = 64
_L = 16
_NC = 2
_NS = 16
_NW = _NC * _NS
_BW = _B // _NW
_CHUNK = 128
_NCHUNK = _BW // _CHUNK


def _gmf_body(uidx_hbm, iidx_hbm, utab_hbm, itab_hbm, w_hbm, b_hbm, out_hbm,
              uidx_v, iidx_v, urows_v, irows_v, w_v, b_v, out_v, sem):
    wid = lax.axis_index("s") * _NC + lax.axis_index("c")
    row0 = wid * _NCHUNK
    base = wid * _BW

    pltpu.sync_copy(uidx_hbm.at[pl.ds(row0, _NCHUNK)], uidx_v)
    pltpu.sync_copy(iidx_hbm.at[pl.ds(row0, _NCHUNK)], iidx_v)
    pltpu.sync_copy(w_hbm, w_v)
    pltpu.sync_copy(b_hbm, b_v)

    copies = []
    for j in range(_NCHUNK):
        copies.append(pltpu.async_copy(
            utab_hbm.at[uidx_v.at[j]],
            urows_v.at[pl.ds(j * _CHUNK, _CHUNK)], sem))
        copies.append(pltpu.async_copy(
            itab_hbm.at[iidx_v.at[j]],
            irows_v.at[pl.ds(j * _CHUNK, _CHUNK)], sem))
    for c in copies:
        c.wait()

    wv = [w_v[pl.ds(k * _L, _L)] for k in range(_F // _L)]
    bias = b_v[...]
    lane = lax.broadcasted_iota(jnp.int32, (_L,), 0)
    perms = [lane ^ (_L >> (p + 1)) for p in range(4)]

    def group(g, carry):
        res = bias
        for r in range(_L):
            b = g * _L + r
            acc = (urows_v[b, pl.ds(0, _L)] * irows_v[b, pl.ds(0, _L)]) * wv[0]
            for k in range(1, _F // _L):
                acc = acc + (urows_v[b, pl.ds(k * _L, _L)]
                             * irows_v[b, pl.ds(k * _L, _L)]) * wv[k]
            for p in perms:
                acc = acc + acc.at[p].get(mode="promise_in_bounds")
            res = jnp.where(lane == r, res + acc, res)
        out_v[pl.ds(g * _L, _L)] = res
        return carry

    lax.fori_loop(0, _BW // _L, group, 0)
    pltpu.sync_copy(out_v, out_hbm.at[pl.ds(base, _BW)])


@jax.jit
def _gmf(user_indices, item_indices, user_table, item_table, linear_w, linear_b):
    uidx = user_indices.astype(jnp.int32).reshape(_B // _CHUNK, _CHUNK)
    iidx = item_indices.astype(jnp.int32).reshape(_B // _CHUNK, _CHUNK)
    w = linear_w.reshape(_F)
    bias = jnp.full((_L,), linear_b[0], dtype=jnp.float32)
    mesh = plsc.VectorSubcoreMesh(core_axis_name="c", subcore_axis_name="s",
                                  num_cores=_NC, num_subcores=_NS)
    out = pl.kernel(
        _gmf_body,
        out_type=jax.ShapeDtypeStruct((_B,), jnp.float32),
        mesh=mesh,
        compiler_params=pltpu.CompilerParams(use_tc_tiling_on_sc=False),
        scratch_types=[
            pltpu.VMEM((_NCHUNK, _CHUNK), jnp.int32),
            pltpu.VMEM((_NCHUNK, _CHUNK), jnp.int32),
            pltpu.VMEM((_BW, _F), jnp.float32),
            pltpu.VMEM((_BW, _F), jnp.float32),
            pltpu.VMEM((_F,), jnp.float32),
            pltpu.VMEM((_L,), jnp.float32),
            pltpu.VMEM((_BW,), jnp.float32),
            pltpu.SemaphoreType.DMA,
        ],
    )(uidx, iidx, user_table, item_table, w, bias)
    return out.reshape(_B, 1)


def kernel(user_indices, item_indices, user_table, item_table, linear_w, linear_b):
    return _gmf(user_indices, item_indices, user_table, item_table,
                linear_w, linear_b)
